# baseline (speedup 1.0000x reference)
import jax
import jax.numpy as jnp
from jax.experimental import pallas as pl

H, C = 8, 16


def _gatv2_conv(x, ei, Wl, bl, Wr, br, att, bias):
    N = x.shape[0]
    src, dst = ei[0], ei[1]
    xl = (x @ Wl + bl).reshape(N, H, C)
    xr = (x @ Wr + br).reshape(N, H, C)
    m = xl[src] + xr[dst]
    e = jnp.sum(jax.nn.leaky_relu(m, 0.2) * att[None], axis=-1)
    emax = jax.ops.segment_max(e, dst, num_segments=N)
    emax = jnp.where(jnp.isfinite(emax), emax, 0.0)
    ex = jnp.exp(e - emax[dst])
    denom = jax.ops.segment_sum(ex, dst, num_segments=N)
    alpha = ex / (denom[dst] + 1e-16)
    out = jax.ops.segment_sum(alpha[:, :, None] * xl[src], dst, num_segments=N)
    return out.reshape(N, H * C) + bias


def _copy_kernel(x_ref, o_ref):
    o_ref[...] = x_ref[...]


def kernel(x, edges_idx, Wl1, bl1, Wr1, br1, att1, b1, Wl2, bl2, Wr2, br2, att2, b2):
    N = x.shape[0]
    loop = jnp.arange(N, dtype=edges_idx.dtype)
    ei = jnp.concatenate([edges_idx, jnp.stack([loop, loop])], axis=1)
    h = jax.nn.relu(_gatv2_conv(x, ei, Wl1, bl1, Wr1, br1, att1, b1))
    out = _gatv2_conv(h, ei, Wl2, bl2, Wr2, br2, att2, b2)
    return pl.pallas_call(
        _copy_kernel,
        out_shape=jax.ShapeDtypeStruct(out.shape, out.dtype),
    )(out)


# speedup vs baseline: 19.9655x; 19.9655x over previous
import functools

import jax
import jax.numpy as jnp
import numpy as np
from jax import lax
from jax.experimental import pallas as pl
from jax.experimental.pallas import tpu as pltpu
from jax.experimental.pallas import tpu_sc as plsc

H, C = 8, 16
D = H * C
N_PAD = 10240
BATCH = 64
NW = 32
NEG = np.float32(-1e30)


def _lane():
    return lax.iota(jnp.int32, 16)


def _take16(v, idx):
    return lax.gather(
        v, idx.reshape(16, 1),
        dimension_numbers=lax.GatherDimensionNumbers(
            offset_dims=(), collapsed_slice_dims=(0,), start_index_map=(0,)),
        slice_sizes=(1,),
        mode=lax.GatherScatterMode.PROMISE_IN_BOUNDS)


def _mm_body(x_ref, wl_ref, bl_ref, wr_ref, br_ref, ol_ref, or_ref):
    xb = x_ref[...]
    ol_ref[...] = jnp.dot(xb, wl_ref[...],
                          preferred_element_type=jnp.float32) + bl_ref[...]
    or_ref[...] = jnp.dot(xb, wr_ref[...],
                          preferred_element_type=jnp.float32) + br_ref[...]


def _mm(x, Wl, bl, Wr, br):
    n = x.shape[0]
    grid = n // 128
    return pl.pallas_call(
        _mm_body,
        grid=(grid,),
        in_specs=[
            pl.BlockSpec((128, D), lambda i: (i, 0)),
            pl.BlockSpec((D, D), lambda i: (0, 0)),
            pl.BlockSpec((1, D), lambda i: (0, 0)),
            pl.BlockSpec((D, D), lambda i: (0, 0)),
            pl.BlockSpec((1, D), lambda i: (0, 0)),
        ],
        out_specs=[
            pl.BlockSpec((128, D), lambda i: (i, 0)),
            pl.BlockSpec((128, D), lambda i: (i, 0)),
        ],
        out_shape=[
            jax.ShapeDtypeStruct((n, D), jnp.float32),
            jax.ShapeDtypeStruct((n, D), jnp.float32),
        ],
    )(x, Wl, bl.reshape(1, D), Wr, br.reshape(1, D))


def _maxcomb(mp):
    npt = N_PAD // NW
    flat = npt * H
    mesh = plsc.VectorSubcoreMesh(core_axis_name="c", subcore_axis_name="s")

    @functools.partial(
        pl.kernel,
        out_type=jax.ShapeDtypeStruct((N_PAD, 16), jnp.float32),
        mesh=mesh,
        scratch_types=[
            pltpu.VMEM((16, flat), jnp.float32),
            pltpu.VMEM((flat,), jnp.float32),
            pltpu.VMEM((npt, 16), jnp.float32),
            pltpu.SemaphoreType.DMA,
        ],
        compiler_params=pltpu.CompilerParams(
            needs_layout_passes=False, use_tc_tiling_on_sc=False),
    )
    def k(mp_h, emax_h, slabs, acc, dump, sem):
        cid = lax.axis_index("c")
        sid = lax.axis_index("s")
        wid = cid * 16 + sid
        off = wid * flat
        lane = _lane()

        for r in range(2):
            cps = [pltpu.async_copy(
                mp_h.at[r * 16 + t, pl.ds(off, flat)], slabs.at[t], sem)
                for t in range(16)]
            for cp in cps:
                cp.wait()

            def mx(i, _, first=(r == 0)):
                v = slabs[0, pl.ds(i * 16, 16)]
                for t in range(1, 16):
                    v = jnp.maximum(v, slabs[t, pl.ds(i * 16, 16)])
                if not first:
                    v = jnp.maximum(v, acc[pl.ds(i * 16, 16)])
                acc[pl.ds(i * 16, 16)] = v
                return 0

            lax.fori_loop(0, flat // 16, mx, 0)

        def expand(i, _):
            v = acc[pl.ds(i * 16, 16)]
            dump[i * 2, pl.ds(0, 16)] = _take16(v, lane & 7)
            dump[i * 2 + 1, pl.ds(0, 16)] = _take16(v, (lane & 7) + 8)
            return 0
        lax.fori_loop(0, flat // 16, expand, 0)

        pltpu.sync_copy(dump, emax_h.at[pl.ds(wid * npt, npt)])

    return k(mp)


def _expander():
    return (jnp.arange(D, dtype=jnp.int32)[None, :] // C
            == jnp.arange(H, dtype=jnp.int32)[:, None]).astype(jnp.float32)


def _fin_mm_body(np_ref, b_ref, wl_ref, bl_ref, wr_ref, br_ref,
                 ol_ref, or_ref):
    s = np_ref[0] + np_ref[1]
    dn = jnp.dot(s[:, 0:8] + jnp.float32(1e-16), _expander(),
                 preferred_element_type=jnp.float32)
    h = jnp.maximum(s[:, 16:144] / dn + b_ref[...], 0.0)
    ol_ref[...] = jnp.dot(h, wl_ref[...],
                          preferred_element_type=jnp.float32) + bl_ref[...]
    or_ref[...] = jnp.dot(h, wr_ref[...],
                          preferred_element_type=jnp.float32) + br_ref[...]


def _fin_mm(numpart, bias, Wl, bl, Wr, br):
    grid = N_PAD // 128
    return pl.pallas_call(
        _fin_mm_body,
        grid=(grid,),
        in_specs=[
            pl.BlockSpec((2, 128, 144), lambda i: (0, i, 0)),
            pl.BlockSpec((1, D), lambda i: (0, 0)),
            pl.BlockSpec((D, D), lambda i: (0, 0)),
            pl.BlockSpec((1, D), lambda i: (0, 0)),
            pl.BlockSpec((D, D), lambda i: (0, 0)),
            pl.BlockSpec((1, D), lambda i: (0, 0)),
        ],
        out_specs=[
            pl.BlockSpec((128, D), lambda i: (i, 0)),
            pl.BlockSpec((128, D), lambda i: (i, 0)),
        ],
        out_shape=[
            jax.ShapeDtypeStruct((N_PAD, D), jnp.float32),
            jax.ShapeDtypeStruct((N_PAD, D), jnp.float32),
        ],
    )(numpart, bias.reshape(1, D), Wl, bl.reshape(1, D), Wr, br.reshape(1, D))


def _fin_body(np_ref, b_ref, o_ref):
    s = np_ref[0] + np_ref[1]
    dn = jnp.dot(s[:, 0:8] + jnp.float32(1e-16), _expander(),
                 preferred_element_type=jnp.float32)
    o_ref[...] = s[:, 16:144] / dn + b_ref[...]


def _fin(numpart, bias):
    grid = N_PAD // 128
    return pl.pallas_call(
        _fin_body,
        grid=(grid,),
        in_specs=[
            pl.BlockSpec((2, 128, 144), lambda i: (0, i, 0)),
            pl.BlockSpec((1, D), lambda i: (0, 0)),
        ],
        out_specs=pl.BlockSpec((128, D), lambda i: (i, 0)),
        out_shape=jax.ShapeDtypeStruct((N_PAD, D), jnp.float32),
    )(numpart, bias.reshape(1, D))


def _pass_a(src, dst, xl, xr, att, n_batches):
    e_per_w = n_batches * BATCH
    mesh = plsc.VectorSubcoreMesh(core_axis_name="c", subcore_axis_name="s")

    @functools.partial(
        pl.kernel,
        out_type=[
            jax.ShapeDtypeStruct((e_per_w * NW, 16), jnp.float32),
            jax.ShapeDtypeStruct((NW, N_PAD * H), jnp.float32),
        ],
        mesh=mesh,
        scratch_types=[
            pltpu.VMEM((BATCH,), jnp.int32),
            pltpu.VMEM((BATCH,), jnp.int32),
            pltpu.VMEM((BATCH, D), jnp.float32),
            pltpu.VMEM((BATCH, D), jnp.float32),
            pltpu.VMEM((BATCH, 16), jnp.float32),
            pltpu.VMEM((H, C), jnp.float32),
            pltpu.VMEM((N_PAD * H,), jnp.float32),
            pltpu.SemaphoreType.DMA,
        ],
        compiler_params=pltpu.CompilerParams(needs_layout_passes=False, use_tc_tiling_on_sc=False),
    )
    def k(src_h, dst_h, xl_h, xr_h, att_h, e_h, maxp_h,
          srcv, dstv, xls, xrd, ebuf, attv, table, sem):
        cid = lax.axis_index("c")
        sid = lax.axis_index("s")
        wid = cid * 16 + sid
        base_w = wid * e_per_w
        lane = _lane()
        negv = lane.astype(jnp.float32) * 0.0 + NEG

        pltpu.sync_copy(att_h, attv)

        def init_t(i, _):
            table[pl.ds(i * 16, 16)] = negv
            return 0
        lax.fori_loop(0, N_PAD * H // 16, init_t, 0)

        def batch_body(g, _):
            base = base_w + g * BATCH
            pltpu.sync_copy(src_h.at[pl.ds(base, BATCH)], srcv)
            pltpu.sync_copy(dst_h.at[pl.ds(base, BATCH)], dstv)
            pltpu.async_copy(xl_h.at[srcv], xls, sem).wait()
            pltpu.async_copy(xr_h.at[dstv], xrd, sem).wait()

            def edge_body(b, _):
                e16 = negv * 0.0
                for h in range(H):
                    v = xls[b, pl.ds(h * C, C)] + xrd[b, pl.ds(h * C, C)]
                    v = jnp.maximum(v, jnp.float32(0.2) * v)
                    s = v * attv[h, pl.ds(0, C)]
                    for sh in (1, 2, 4, 8):
                        s = s + _take16(s, lane ^ sh)
                    e16 = jnp.where((lane & 7) == h, s, e16)
                ebuf[b, pl.ds(0, 16)] = e16
                return 0
            lax.fori_loop(0, BATCH, edge_body, 0)

            pltpu.sync_copy(ebuf, e_h.at[pl.ds(base, BATCH)])

            lt8 = lane < 8

            def group_body(q, _):
                dv = dstv[pl.ds(q * 16, 16)]
                for p in range(8):
                    b0 = q * 16 + 2 * p
                    d0 = dv[2 * p]
                    d1 = dv[2 * p + 1]
                    e0 = ebuf[b0, pl.ds(0, 16)]
                    e1 = ebuf[b0 + 1, pl.ds(0, 16)]
                    cmb = jnp.where(lt8, e0, e1)
                    idx = jnp.where(lt8, d0 * H, d1 * H) + (lane & 7)
                    eq = d0 == d1
                    sw = _take16(cmb, lane ^ 8)
                    cmb = jnp.where(eq, jnp.maximum(cmb, sw), cmb)
                    cur = plsc.load_gather(table, [idx])
                    plsc.store_scatter(table, [idx], jnp.maximum(cur, cmb))
                return 0
            lax.fori_loop(0, BATCH // 16, group_body, 0)
            return 0

        lax.fori_loop(0, n_batches, batch_body, 0)

        pltpu.sync_copy(table, maxp_h.at[wid])

    return k(src, dst, xl, xr, att)


def _pass_b(src, dst, e, emax, xl, n_batches):
    e_per_w = n_batches * BATCH
    mesh = plsc.VectorSubcoreMesh(core_axis_name="c", subcore_axis_name="s")

    @functools.partial(
        pl.kernel,
        out_type=jax.ShapeDtypeStruct((2, N_PAD, 144), jnp.float32),
        mesh=mesh,
        scratch_types=[
            pltpu.VMEM((BATCH,), jnp.int32),
            pltpu.VMEM((BATCH,), jnp.int32),
            pltpu.VMEM((BATCH, D), jnp.float32),
            pltpu.VMEM((BATCH, 16), jnp.float32),
            pltpu.VMEM((BATCH, 16), jnp.float32),
            pltpu.VMEM((BATCH, 144), jnp.float32),
            pltpu.VMEM_SHARED((N_PAD, 144), jnp.float32),
            pltpu.SemaphoreType.DMA,
        ],
        compiler_params=pltpu.CompilerParams(needs_layout_passes=False, use_tc_tiling_on_sc=False),
    )
    def k(src_h, dst_h, e_h, emax_h, xl_h, nump_h,
          srcv, dstv, xls, ev, emv, contrib, shared, sem):
        cid = lax.axis_index("c")
        sid = lax.axis_index("s")
        wid = cid * 16 + sid
        base_w = wid * e_per_w
        npt = N_PAD // 16
        lane = _lane()
        zerov = lane.astype(jnp.float32) * 0.0

        def zero_rows(i, _):
            r = i // 9
            col = (i % 9) * 16
            contrib[r, pl.ds(col, 16)] = zerov
            return 0
        lax.fori_loop(0, BATCH * 9, zero_rows, 0)

        for j in range(npt // BATCH):
            pltpu.sync_copy(
                contrib, shared.at[pl.ds(sid * npt + j * BATCH, BATCH)])
        plsc.subcore_barrier()

        def batch_body(g, _):
            base = base_w + g * BATCH
            pltpu.sync_copy(src_h.at[pl.ds(base, BATCH)], srcv)
            pltpu.sync_copy(dst_h.at[pl.ds(base, BATCH)], dstv)
            pltpu.async_copy(xl_h.at[srcv], xls, sem).wait()
            pltpu.async_copy(emax_h.at[dstv], emv, sem).wait()
            pltpu.sync_copy(e_h.at[pl.ds(base, BATCH)], ev)

            def edge_body(b, _):
                ex = jnp.exp(ev[b, pl.ds(0, 16)] - emv[b, pl.ds(0, 16)])
                contrib[b, pl.ds(0, 16)] = ex
                for h in range(H):
                    sp = _take16(ex, (lane & 0) + h)
                    contrib[b, pl.ds(16 + h * C, C)] = (
                        sp * xls[b, pl.ds(h * C, C)])
                return 0
            lax.fori_loop(0, BATCH, edge_body, 0)

            pltpu.async_copy(contrib, shared.at[dstv], sem, add=True).wait()
            return 0

        lax.fori_loop(0, n_batches, batch_body, 0)
        plsc.subcore_barrier()

        for j in range(npt // BATCH):
            row = sid * npt + j * BATCH
            pltpu.sync_copy(shared.at[pl.ds(row, BATCH)], contrib)
            pltpu.sync_copy(contrib, nump_h.at[cid, pl.ds(row, BATCH)])

    return k(src, dst, e, emax, xl)


def kernel(x, edges_idx, Wl1, bl1, Wr1, br1, att1, b1,
           Wl2, bl2, Wr2, br2, att2, b2):
    N = x.shape[0]
    E = edges_idx.shape[1]
    e_tot = E + N
    per_round = NW * BATCH
    n_batches = -(-e_tot // per_round)
    e_pad = n_batches * per_round

    loop = jnp.arange(N, dtype=edges_idx.dtype)
    ei = jnp.concatenate([edges_idx, jnp.stack([loop, loop])], axis=1)
    pad = jnp.full((2, e_pad - e_tot), N, dtype=edges_idx.dtype)
    ei = jnp.concatenate([ei, pad], axis=1)
    src, dst = ei[0], ei[1]

    xp = jnp.zeros((N_PAD, D), jnp.float32).at[:N].set(x)

    xl1, xr1 = _mm(xp, Wl1, bl1, Wr1, br1)
    e1, maxp1 = _pass_a(src, dst, xl1, xr1, att1, n_batches)
    emax1 = _maxcomb(maxp1)
    nump1 = _pass_b(src, dst, e1, emax1, xl1, n_batches)
    xl2, xr2 = _fin_mm(nump1, b1, Wl2, bl2, Wr2, br2)

    e2, maxp2 = _pass_a(src, dst, xl2, xr2, att2, n_batches)
    emax2 = _maxcomb(maxp2)
    nump2 = _pass_b(src, dst, e2, emax2, xl2, n_batches)
    out = _fin(nump2, b2)

    return out[:N]


# speedup vs baseline: 26.6638x; 1.3355x over previous
import functools

import jax
import jax.numpy as jnp
import numpy as np
from jax import lax
from jax.experimental import pallas as pl
from jax.experimental.pallas import tpu as pltpu
from jax.experimental.pallas import tpu_sc as plsc

H, C = 8, 16
D = H * C
N_PAD = 10240
BATCH = 64
NW = 32
NEG = np.float32(-1e30)


def _lane():
    return lax.iota(jnp.int32, 16)


def _take16(v, idx):
    return lax.gather(
        v, idx.reshape(16, 1),
        dimension_numbers=lax.GatherDimensionNumbers(
            offset_dims=(), collapsed_slice_dims=(0,), start_index_map=(0,)),
        slice_sizes=(1,),
        mode=lax.GatherScatterMode.PROMISE_IN_BOUNDS)


def _mm_body(x_ref, wl_ref, bl_ref, wr_ref, br_ref, ol_ref, or_ref):
    xb = x_ref[...]
    ol_ref[...] = jnp.dot(xb, wl_ref[...],
                          preferred_element_type=jnp.float32) + bl_ref[...]
    or_ref[...] = jnp.dot(xb, wr_ref[...],
                          preferred_element_type=jnp.float32) + br_ref[...]


def _mm(x, Wl, bl, Wr, br):
    n = x.shape[0]
    grid = n // 128
    return pl.pallas_call(
        _mm_body,
        grid=(grid,),
        in_specs=[
            pl.BlockSpec((128, D), lambda i: (i, 0)),
            pl.BlockSpec((D, D), lambda i: (0, 0)),
            pl.BlockSpec((1, D), lambda i: (0, 0)),
            pl.BlockSpec((D, D), lambda i: (0, 0)),
            pl.BlockSpec((1, D), lambda i: (0, 0)),
        ],
        out_specs=[
            pl.BlockSpec((128, D), lambda i: (i, 0)),
            pl.BlockSpec((128, D), lambda i: (i, 0)),
        ],
        out_shape=[
            jax.ShapeDtypeStruct((n, D), jnp.float32),
            jax.ShapeDtypeStruct((n, D), jnp.float32),
        ],
    )(x, Wl, bl.reshape(1, D), Wr, br.reshape(1, D))


def _maxcomb(mp):
    npt = N_PAD // NW
    flat = npt * H
    mesh = plsc.VectorSubcoreMesh(core_axis_name="c", subcore_axis_name="s")

    @functools.partial(
        pl.kernel,
        out_type=jax.ShapeDtypeStruct((N_PAD, 16), jnp.float32),
        mesh=mesh,
        scratch_types=[
            pltpu.VMEM((16, flat), jnp.float32),
            pltpu.VMEM((flat,), jnp.float32),
            pltpu.VMEM((npt, 16), jnp.float32),
            pltpu.SemaphoreType.DMA,
        ],
        compiler_params=pltpu.CompilerParams(
            needs_layout_passes=False, use_tc_tiling_on_sc=False),
    )
    def k(mp_h, emax_h, slabs, acc, dump, sem):
        cid = lax.axis_index("c")
        sid = lax.axis_index("s")
        wid = cid * 16 + sid
        off = wid * flat
        lane = _lane()

        for r in range(2):
            cps = [pltpu.async_copy(
                mp_h.at[r * 16 + t, pl.ds(off, flat)], slabs.at[t], sem)
                for t in range(16)]
            for cp in cps:
                cp.wait()

            def mx(i, _, first=(r == 0)):
                v = slabs[0, pl.ds(i * 16, 16)]
                for t in range(1, 16):
                    v = jnp.maximum(v, slabs[t, pl.ds(i * 16, 16)])
                if not first:
                    v = jnp.maximum(v, acc[pl.ds(i * 16, 16)])
                acc[pl.ds(i * 16, 16)] = v
                return 0

            lax.fori_loop(0, flat // 16, mx, 0)

        def expand(i, _):
            v = acc[pl.ds(i * 16, 16)]
            dump[i * 2, pl.ds(0, 16)] = _take16(v, lane & 7)
            dump[i * 2 + 1, pl.ds(0, 16)] = _take16(v, (lane & 7) + 8)
            return 0
        lax.fori_loop(0, flat // 16, expand, 0)

        pltpu.sync_copy(dump, emax_h.at[pl.ds(wid * npt, npt)])

    return k(mp)


def _expander():
    return (jnp.arange(D, dtype=jnp.int32)[None, :] // C
            == jnp.arange(H, dtype=jnp.int32)[:, None]).astype(jnp.float32)


def _fin_mm_body(np_ref, b_ref, wl_ref, bl_ref, wr_ref, br_ref,
                 ol_ref, or_ref):
    s = np_ref[0] + np_ref[1]
    dn = jnp.dot(s[:, 0:8] + jnp.float32(1e-16), _expander(),
                 preferred_element_type=jnp.float32)
    h = jnp.maximum(s[:, 16:144] / dn + b_ref[...], 0.0)
    ol_ref[...] = jnp.dot(h, wl_ref[...],
                          preferred_element_type=jnp.float32) + bl_ref[...]
    or_ref[...] = jnp.dot(h, wr_ref[...],
                          preferred_element_type=jnp.float32) + br_ref[...]


def _fin_mm(numpart, bias, Wl, bl, Wr, br):
    grid = N_PAD // 128
    return pl.pallas_call(
        _fin_mm_body,
        grid=(grid,),
        in_specs=[
            pl.BlockSpec((2, 128, 144), lambda i: (0, i, 0)),
            pl.BlockSpec((1, D), lambda i: (0, 0)),
            pl.BlockSpec((D, D), lambda i: (0, 0)),
            pl.BlockSpec((1, D), lambda i: (0, 0)),
            pl.BlockSpec((D, D), lambda i: (0, 0)),
            pl.BlockSpec((1, D), lambda i: (0, 0)),
        ],
        out_specs=[
            pl.BlockSpec((128, D), lambda i: (i, 0)),
            pl.BlockSpec((128, D), lambda i: (i, 0)),
        ],
        out_shape=[
            jax.ShapeDtypeStruct((N_PAD, D), jnp.float32),
            jax.ShapeDtypeStruct((N_PAD, D), jnp.float32),
        ],
    )(numpart, bias.reshape(1, D), Wl, bl.reshape(1, D), Wr, br.reshape(1, D))


def _fin_body(np_ref, b_ref, o_ref):
    s = np_ref[0] + np_ref[1]
    dn = jnp.dot(s[:, 0:8] + jnp.float32(1e-16), _expander(),
                 preferred_element_type=jnp.float32)
    o_ref[...] = s[:, 16:144] / dn + b_ref[...]


def _fin(numpart, bias):
    grid = N_PAD // 128
    return pl.pallas_call(
        _fin_body,
        grid=(grid,),
        in_specs=[
            pl.BlockSpec((2, 128, 144), lambda i: (0, i, 0)),
            pl.BlockSpec((1, D), lambda i: (0, 0)),
        ],
        out_specs=pl.BlockSpec((128, D), lambda i: (i, 0)),
        out_shape=jax.ShapeDtypeStruct((N_PAD, D), jnp.float32),
    )(numpart, bias.reshape(1, D))


def _pass_a(src, dst, xl, xr, att, e_per_w):
    BA = 128
    n_batches = e_per_w // BA
    mesh = plsc.VectorSubcoreMesh(core_axis_name="c", subcore_axis_name="s")

    @functools.partial(
        pl.kernel,
        out_type=[
            jax.ShapeDtypeStruct((e_per_w * NW, 16), jnp.float32),
            jax.ShapeDtypeStruct((NW, N_PAD * H), jnp.float32),
        ],
        mesh=mesh,
        scratch_types=[
            pltpu.VMEM((BA,), jnp.int32),
            pltpu.VMEM((BA,), jnp.int32),
            pltpu.VMEM((BA, D), jnp.float32),
            pltpu.VMEM((BA, D), jnp.float32),
            pltpu.VMEM((BA, 16), jnp.float32),
            pltpu.VMEM((H, C), jnp.float32),
            pltpu.VMEM((N_PAD * H,), jnp.float32),
            pltpu.SemaphoreType.DMA,
        ],
        compiler_params=pltpu.CompilerParams(needs_layout_passes=False, use_tc_tiling_on_sc=False),
    )
    def k(src_h, dst_h, xl_h, xr_h, att_h, e_h, maxp_h,
          srcv, dstv, xls, xrd, ebuf, attv, table, sem):
        cid = lax.axis_index("c")
        sid = lax.axis_index("s")
        wid = cid * 16 + sid
        base_w = wid * e_per_w
        lane = _lane()
        negv = lane.astype(jnp.float32) * 0.0 + NEG

        pltpu.sync_copy(att_h, attv)

        def init_t(i, _):
            table[pl.ds(i * 16, 16)] = negv
            return 0
        lax.fori_loop(0, N_PAD * H // 16, init_t, 0)

        def batch_body(g, _):
            base = base_w + g * BA
            c1 = pltpu.async_copy(src_h.at[pl.ds(base, BA)], srcv, sem)
            c2 = pltpu.async_copy(dst_h.at[pl.ds(base, BA)], dstv, sem)
            c1.wait()
            c2.wait()
            c3 = pltpu.async_copy(xl_h.at[srcv], xls, sem)
            c4 = pltpu.async_copy(xr_h.at[dstv], xrd, sem)
            c3.wait()
            c4.wait()

            def edge_body(b, _):
                e16 = negv * 0.0
                for h in range(H):
                    v = xls[b, pl.ds(h * C, C)] + xrd[b, pl.ds(h * C, C)]
                    v = jnp.maximum(v, jnp.float32(0.2) * v)
                    s = v * attv[h, pl.ds(0, C)]
                    for sh in (1, 2, 4, 8):
                        s = s + _take16(s, lane ^ sh)
                    e16 = jnp.where((lane & 7) == h, s, e16)
                ebuf[b, pl.ds(0, 16)] = e16
                return 0
            lax.fori_loop(0, BA, edge_body, 0)

            pltpu.sync_copy(ebuf, e_h.at[pl.ds(base, BA)])

            lt8 = lane < 8

            def group_body(q, _):
                dv = dstv[pl.ds(q * 16, 16)]
                for p in range(8):
                    b0 = q * 16 + 2 * p
                    d0 = dv[2 * p]
                    d1 = dv[2 * p + 1]
                    e0 = ebuf[b0, pl.ds(0, 16)]
                    e1 = ebuf[b0 + 1, pl.ds(0, 16)]
                    cmb = jnp.where(lt8, e0, e1)
                    idx = jnp.where(lt8, d0 * H, d1 * H) + (lane & 7)
                    eq = d0 == d1
                    sw = _take16(cmb, lane ^ 8)
                    cmb = jnp.where(eq, jnp.maximum(cmb, sw), cmb)
                    cur = plsc.load_gather(table, [idx])
                    plsc.store_scatter(table, [idx], jnp.maximum(cur, cmb))
                return 0
            lax.fori_loop(0, BA // 16, group_body, 0)
            return 0

        lax.fori_loop(0, n_batches, batch_body, 0)

        pltpu.sync_copy(table, maxp_h.at[wid])

    return k(src, dst, xl, xr, att)


def _pass_b(src, dst, e, emax, xl, e_per_w):
    BB = 96
    n_batches = e_per_w // BB
    mesh = plsc.VectorSubcoreMesh(core_axis_name="c", subcore_axis_name="s")

    @functools.partial(
        pl.kernel,
        out_type=jax.ShapeDtypeStruct((2, N_PAD, 144), jnp.float32),
        mesh=mesh,
        scratch_types=[
            pltpu.VMEM((BB,), jnp.int32),
            pltpu.VMEM((BB,), jnp.int32),
            pltpu.VMEM((BB, D), jnp.float32),
            pltpu.VMEM((BB, 16), jnp.float32),
            pltpu.VMEM((BB, 16), jnp.float32),
            pltpu.VMEM((BB, 144), jnp.float32),
            pltpu.VMEM_SHARED((N_PAD, 144), jnp.float32),
            pltpu.SemaphoreType.DMA,
        ],
        compiler_params=pltpu.CompilerParams(needs_layout_passes=False, use_tc_tiling_on_sc=False),
    )
    def k(src_h, dst_h, e_h, emax_h, xl_h, nump_h,
          srcv, dstv, xls, ev, emv, contrib, shared, sem):
        cid = lax.axis_index("c")
        sid = lax.axis_index("s")
        wid = cid * 16 + sid
        base_w = wid * e_per_w
        npt = N_PAD // 16
        lane = _lane()
        zerov = lane.astype(jnp.float32) * 0.0

        def zero_rows(i, _):
            r = i // 9
            col = (i % 9) * 16
            contrib[r, pl.ds(col, 16)] = zerov
            return 0
        lax.fori_loop(0, 64 * 9, zero_rows, 0)

        for j in range(npt // 64):
            pltpu.sync_copy(
                contrib.at[pl.ds(0, 64)],
                shared.at[pl.ds(sid * npt + j * 64, 64)])
        plsc.subcore_barrier()

        def batch_body(g, _):
            base = base_w + g * BB
            c1 = pltpu.async_copy(src_h.at[pl.ds(base, BB)], srcv, sem)
            c2 = pltpu.async_copy(dst_h.at[pl.ds(base, BB)], dstv, sem)
            c3 = pltpu.async_copy(e_h.at[pl.ds(base, BB)], ev, sem)
            c1.wait()
            c2.wait()
            c3.wait()
            c4 = pltpu.async_copy(xl_h.at[srcv], xls, sem)
            c5 = pltpu.async_copy(emax_h.at[dstv], emv, sem)
            c4.wait()
            c5.wait()

            def edge_body(b, _):
                ex = jnp.exp(ev[b, pl.ds(0, 16)] - emv[b, pl.ds(0, 16)])
                contrib[b, pl.ds(0, 16)] = ex
                for h in range(H):
                    sp = _take16(ex, (lane & 0) + h)
                    contrib[b, pl.ds(16 + h * C, C)] = (
                        sp * xls[b, pl.ds(h * C, C)])
                return 0
            lax.fori_loop(0, BB, edge_body, 0)

            pltpu.async_copy(contrib, shared.at[dstv], sem, add=True).wait()
            return 0

        lax.fori_loop(0, n_batches, batch_body, 0)
        plsc.subcore_barrier()

        for j in range(npt // 64):
            row = sid * npt + j * 64
            pltpu.sync_copy(shared.at[pl.ds(row, 64)],
                            contrib.at[pl.ds(0, 64)])
            pltpu.sync_copy(contrib.at[pl.ds(0, 64)],
                            nump_h.at[cid, pl.ds(row, 64)])

    return k(src, dst, e, emax, xl)


def kernel(x, edges_idx, Wl1, bl1, Wr1, br1, att1, b1,
           Wl2, bl2, Wr2, br2, att2, b2):
    N = x.shape[0]
    E = edges_idx.shape[1]
    e_tot = E + N
    per_round = NW * 384
    e_pad = -(-e_tot // per_round) * per_round
    e_per_w = e_pad // NW

    loop = jnp.arange(N, dtype=edges_idx.dtype)
    ei = jnp.concatenate([edges_idx, jnp.stack([loop, loop])], axis=1)
    pad = jnp.full((2, e_pad - e_tot), N, dtype=edges_idx.dtype)
    ei = jnp.concatenate([ei, pad], axis=1)
    src, dst = ei[0], ei[1]

    xp = jnp.zeros((N_PAD, D), jnp.float32).at[:N].set(x)

    xl1, xr1 = _mm(xp, Wl1, bl1, Wr1, br1)
    e1, maxp1 = _pass_a(src, dst, xl1, xr1, att1, e_per_w)
    emax1 = _maxcomb(maxp1)
    nump1 = _pass_b(src, dst, e1, emax1, xl1, e_per_w)
    xl2, xr2 = _fin_mm(nump1, b1, Wl2, bl2, Wr2, br2)

    e2, maxp2 = _pass_a(src, dst, xl2, xr2, att2, e_per_w)
    emax2 = _maxcomb(maxp2)
    nump2 = _pass_b(src, dst, e2, emax2, xl2, e_per_w)
    out = _fin(nump2, b2)

    return out[:N]


# speedup vs baseline: 27.6965x; 1.0387x over previous
import functools

import jax
import jax.numpy as jnp
import numpy as np
from jax import lax
from jax.experimental import pallas as pl
from jax.experimental.pallas import tpu as pltpu
from jax.experimental.pallas import tpu_sc as plsc

H, C = 8, 16
D = H * C
N_PAD = 10240
BATCH = 64
NW = 32
NEG = np.float32(-1e30)


def _lane():
    return lax.iota(jnp.int32, 16)


def _take16(v, idx):
    return lax.gather(
        v, idx.reshape(16, 1),
        dimension_numbers=lax.GatherDimensionNumbers(
            offset_dims=(), collapsed_slice_dims=(0,), start_index_map=(0,)),
        slice_sizes=(1,),
        mode=lax.GatherScatterMode.PROMISE_IN_BOUNDS)


def _mm_body(x_ref, wl_ref, bl_ref, wr_ref, br_ref, ol_ref, or_ref):
    xb = x_ref[...]
    ol_ref[...] = jnp.dot(xb, wl_ref[...],
                          preferred_element_type=jnp.float32) + bl_ref[...]
    or_ref[...] = jnp.dot(xb, wr_ref[...],
                          preferred_element_type=jnp.float32) + br_ref[...]


def _mm(x, Wl, bl, Wr, br):
    n = x.shape[0]
    grid = n // 128
    return pl.pallas_call(
        _mm_body,
        grid=(grid,),
        in_specs=[
            pl.BlockSpec((128, D), lambda i: (i, 0)),
            pl.BlockSpec((D, D), lambda i: (0, 0)),
            pl.BlockSpec((1, D), lambda i: (0, 0)),
            pl.BlockSpec((D, D), lambda i: (0, 0)),
            pl.BlockSpec((1, D), lambda i: (0, 0)),
        ],
        out_specs=[
            pl.BlockSpec((128, D), lambda i: (i, 0)),
            pl.BlockSpec((128, D), lambda i: (i, 0)),
        ],
        out_shape=[
            jax.ShapeDtypeStruct((n, D), jnp.float32),
            jax.ShapeDtypeStruct((n, D), jnp.float32),
        ],
    )(x, Wl, bl.reshape(1, D), Wr, br.reshape(1, D))


def _maxcomb(mp):
    npt = N_PAD // NW
    flat = npt * H
    mesh = plsc.VectorSubcoreMesh(core_axis_name="c", subcore_axis_name="s")

    @functools.partial(
        pl.kernel,
        out_type=jax.ShapeDtypeStruct((N_PAD, 16), jnp.float32),
        mesh=mesh,
        scratch_types=[
            pltpu.VMEM((16, flat), jnp.float32),
            pltpu.VMEM((flat,), jnp.float32),
            pltpu.VMEM((npt, 16), jnp.float32),
            pltpu.SemaphoreType.DMA,
        ],
        compiler_params=pltpu.CompilerParams(
            needs_layout_passes=False, use_tc_tiling_on_sc=False),
    )
    def k(mp_h, emax_h, slabs, acc, dump, sem):
        cid = lax.axis_index("c")
        sid = lax.axis_index("s")
        wid = cid * 16 + sid
        off = wid * flat
        lane = _lane()

        for r in range(2):
            cps = [pltpu.async_copy(
                mp_h.at[r * 16 + t, pl.ds(off, flat)], slabs.at[t], sem)
                for t in range(16)]
            for cp in cps:
                cp.wait()

            def mx(i, _, first=(r == 0)):
                v = slabs[0, pl.ds(i * 16, 16)]
                for t in range(1, 16):
                    v = jnp.maximum(v, slabs[t, pl.ds(i * 16, 16)])
                if not first:
                    v = jnp.maximum(v, acc[pl.ds(i * 16, 16)])
                acc[pl.ds(i * 16, 16)] = v
                return 0

            lax.fori_loop(0, flat // 16, mx, 0)

        def expand(i, _):
            v = acc[pl.ds(i * 16, 16)]
            dump[i * 2, pl.ds(0, 16)] = _take16(v, lane & 7)
            dump[i * 2 + 1, pl.ds(0, 16)] = _take16(v, (lane & 7) + 8)
            return 0
        lax.fori_loop(0, flat // 16, expand, 0)

        pltpu.sync_copy(dump, emax_h.at[pl.ds(wid * npt, npt)])

    return k(mp)


def _expander():
    return (jnp.arange(D, dtype=jnp.int32)[None, :] // C
            == jnp.arange(H, dtype=jnp.int32)[:, None]).astype(jnp.float32)


def _fin_mm_body(np_ref, b_ref, wl_ref, bl_ref, wr_ref, br_ref,
                 ol_ref, or_ref):
    s = np_ref[0] + np_ref[1]
    dn = jnp.dot(s[:, 0:8] + jnp.float32(1e-16), _expander(),
                 preferred_element_type=jnp.float32)
    h = jnp.maximum(s[:, 16:144] / dn + b_ref[...], 0.0)
    ol_ref[...] = jnp.dot(h, wl_ref[...],
                          preferred_element_type=jnp.float32) + bl_ref[...]
    or_ref[...] = jnp.dot(h, wr_ref[...],
                          preferred_element_type=jnp.float32) + br_ref[...]


def _fin_mm(numpart, bias, Wl, bl, Wr, br):
    grid = N_PAD // 128
    return pl.pallas_call(
        _fin_mm_body,
        grid=(grid,),
        in_specs=[
            pl.BlockSpec((2, 128, 144), lambda i: (0, i, 0)),
            pl.BlockSpec((1, D), lambda i: (0, 0)),
            pl.BlockSpec((D, D), lambda i: (0, 0)),
            pl.BlockSpec((1, D), lambda i: (0, 0)),
            pl.BlockSpec((D, D), lambda i: (0, 0)),
            pl.BlockSpec((1, D), lambda i: (0, 0)),
        ],
        out_specs=[
            pl.BlockSpec((128, D), lambda i: (i, 0)),
            pl.BlockSpec((128, D), lambda i: (i, 0)),
        ],
        out_shape=[
            jax.ShapeDtypeStruct((N_PAD, D), jnp.float32),
            jax.ShapeDtypeStruct((N_PAD, D), jnp.float32),
        ],
    )(numpart, bias.reshape(1, D), Wl, bl.reshape(1, D), Wr, br.reshape(1, D))


def _fin_body(np_ref, b_ref, o_ref):
    s = np_ref[0] + np_ref[1]
    dn = jnp.dot(s[:, 0:8] + jnp.float32(1e-16), _expander(),
                 preferred_element_type=jnp.float32)
    o_ref[...] = s[:, 16:144] / dn + b_ref[...]


def _fin(numpart, bias):
    grid = N_PAD // 128
    return pl.pallas_call(
        _fin_body,
        grid=(grid,),
        in_specs=[
            pl.BlockSpec((2, 128, 144), lambda i: (0, i, 0)),
            pl.BlockSpec((1, D), lambda i: (0, 0)),
        ],
        out_specs=pl.BlockSpec((128, D), lambda i: (i, 0)),
        out_shape=jax.ShapeDtypeStruct((N_PAD, D), jnp.float32),
    )(numpart, bias.reshape(1, D))


def _pass_a(src, dst, xl, xr, att, e_per_w):
    BA = 64
    nb = e_per_w // BA
    mesh = plsc.VectorSubcoreMesh(core_axis_name="c", subcore_axis_name="s")

    @functools.partial(
        pl.kernel,
        out_type=[
            jax.ShapeDtypeStruct((e_per_w * NW, 16), jnp.float32),
            jax.ShapeDtypeStruct((NW, N_PAD * H), jnp.float32),
        ],
        mesh=mesh,
        scratch_types=[
            pltpu.VMEM((2, BA), jnp.int32),
            pltpu.VMEM((2, BA), jnp.int32),
            pltpu.VMEM((2, BA, D), jnp.float32),
            pltpu.VMEM((2, BA, D), jnp.float32),
            pltpu.VMEM((2, BA, 16), jnp.float32),
            pltpu.VMEM((H, C), jnp.float32),
            pltpu.VMEM((N_PAD * H,), jnp.float32),
            pltpu.SemaphoreType.DMA,
            pltpu.SemaphoreType.DMA,
            pltpu.SemaphoreType.DMA,
            pltpu.SemaphoreType.DMA,
            pltpu.SemaphoreType.DMA,
            pltpu.SemaphoreType.DMA,
        ],
        compiler_params=pltpu.CompilerParams(needs_layout_passes=False, use_tc_tiling_on_sc=False),
    )
    def k(src_h, dst_h, xl_h, xr_h, att_h, e_h, maxp_h,
          srcv, dstv, xls, xrd, ebuf, attv, table,
          semA0, semA1, semB0, semB1, semE0, semE1):
        semA = (semA0, semA1)
        semB = (semB0, semB1)
        semE = (semE0, semE1)
        cid = lax.axis_index("c")
        sid = lax.axis_index("s")
        wid = cid * 16 + sid
        base_w = wid * e_per_w
        lane = _lane()
        negv = lane.astype(jnp.float32) * 0.0 + NEG

        pltpu.sync_copy(att_h, attv)

        def init_t(i, _):
            table[pl.ds(i * 16, 16)] = negv
            return 0
        lax.fori_loop(0, N_PAD * H // 16, init_t, 0)

        def idx_issue(b, g):
            pltpu.async_copy(src_h.at[pl.ds(base_w + g * BA, BA)],
                             srcv.at[b], semA[b])
            pltpu.async_copy(dst_h.at[pl.ds(base_w + g * BA, BA)],
                             dstv.at[b], semA[b])

        def idx_wait(b, g):
            pltpu.make_async_copy(src_h.at[pl.ds(base_w + g * BA, BA)],
                                  srcv.at[b], semA[b]).wait()
            pltpu.make_async_copy(dst_h.at[pl.ds(base_w + g * BA, BA)],
                                  dstv.at[b], semA[b]).wait()

        def gath_issue(b):
            pltpu.async_copy(xl_h.at[srcv.at[b]], xls.at[b], semB[b])
            pltpu.async_copy(xr_h.at[dstv.at[b]], xrd.at[b], semB[b])

        def gath_wait(b):
            pltpu.make_async_copy(xl_h.at[srcv.at[b]], xls.at[b],
                                  semB[b]).wait()
            pltpu.make_async_copy(xr_h.at[dstv.at[b]], xrd.at[b],
                                  semB[b]).wait()

        def estore_issue(b, g):
            pltpu.async_copy(ebuf.at[b],
                             e_h.at[pl.ds(base_w + g * BA, BA)], semE[b])

        def estore_wait(b, g):
            pltpu.make_async_copy(ebuf.at[b],
                                  e_h.at[pl.ds(base_w + g * BA, BA)],
                                  semE[b]).wait()

        pltpu.sync_copy(src_h.at[pl.ds(base_w, BA)], srcv.at[0])
        pltpu.sync_copy(dst_h.at[pl.ds(base_w, BA)], dstv.at[0])
        pltpu.sync_copy(src_h.at[pl.ds(base_w + BA, BA)], srcv.at[1])
        pltpu.sync_copy(dst_h.at[pl.ds(base_w + BA, BA)], dstv.at[1])
        gath_issue(0)
        gath_issue(1)

        lt8 = lane < 8

        def pair_loop(b):
            def group_body(q, _):
                dv = dstv[b, pl.ds(q * 16, 16)]
                for p in range(8):
                    b0 = q * 16 + 2 * p
                    d0 = dv[2 * p]
                    d1 = dv[2 * p + 1]
                    e0 = ebuf[b, b0, pl.ds(0, 16)]
                    e1 = ebuf[b, b0 + 1, pl.ds(0, 16)]
                    cmb = jnp.where(lt8, e0, e1)
                    idx = jnp.where(lt8, d0 * H, d1 * H) + (lane & 7)
                    eq = d0 == d1
                    sw = _take16(cmb, lane ^ 8)
                    cmb = jnp.where(eq, jnp.maximum(cmb, sw), cmb)
                    cur = plsc.load_gather(table, [idx])
                    plsc.store_scatter(table, [idx], jnp.maximum(cur, cmb))
                return 0
            lax.fori_loop(0, BA // 16, group_body, 0)

        def compute(b):
            def edge_body(eb, _):
                e16 = negv * 0.0
                for h in range(H):
                    v = (xls[b, eb, pl.ds(h * C, C)]
                         + xrd[b, eb, pl.ds(h * C, C)])
                    v = jnp.maximum(v, jnp.float32(0.2) * v)
                    s = v * attv[h, pl.ds(0, C)]
                    for sh in (1, 2, 4, 8):
                        s = s + _take16(s, lane ^ sh)
                    e16 = jnp.where((lane & 7) == h, s, e16)
                ebuf[b, eb, pl.ds(0, 16)] = e16
                return 0
            lax.fori_loop(0, BA, edge_body, 0)

        def outer(gg, _):
            for b in (0, 1):
                g = gg * 2 + b
                gath_wait(b)

                @pl.when(gg >= 1)
                def _():
                    estore_wait(b, g - 2)

                compute(b)
                estore_issue(b, g)
                pair_loop(b)

                @pl.when(g + 2 < nb)
                def _():
                    idx_issue(b, g + 2)

                bb = 1 - b

                @pl.when(jnp.logical_and(g >= 1, g + 1 < nb))
                def _():
                    idx_wait(bb, g + 1)
                    gath_issue(bb)
            return 0

        lax.fori_loop(0, nb // 2, outer, 0)

        estore_wait(0, nb - 2)
        estore_wait(1, nb - 1)

        pltpu.sync_copy(table, maxp_h.at[wid])

    return k(src, dst, xl, xr, att)


def _pass_b(src, dst, e, emax, xl, e_per_w):
    BB = 48
    nb = e_per_w // BB
    mesh = plsc.VectorSubcoreMesh(core_axis_name="c", subcore_axis_name="s")

    @functools.partial(
        pl.kernel,
        out_type=jax.ShapeDtypeStruct((2, N_PAD, 144), jnp.float32),
        mesh=mesh,
        scratch_types=[
            pltpu.VMEM((2, BB), jnp.int32),
            pltpu.VMEM((2, BB), jnp.int32),
            pltpu.VMEM((2, BB), jnp.int32),
            pltpu.VMEM((2, BB, D), jnp.float32),
            pltpu.VMEM((2, BB, 16), jnp.float32),
            pltpu.VMEM((2, BB, 16), jnp.float32),
            pltpu.VMEM((2, BB, 144), jnp.float32),
            pltpu.VMEM_SHARED((N_PAD, 144), jnp.float32),
            pltpu.SemaphoreType.DMA,
            pltpu.SemaphoreType.DMA,
            pltpu.SemaphoreType.DMA,
            pltpu.SemaphoreType.DMA,
            pltpu.SemaphoreType.DMA,
            pltpu.SemaphoreType.DMA,
        ],
        compiler_params=pltpu.CompilerParams(needs_layout_passes=False, use_tc_tiling_on_sc=False),
    )
    def k(src_h, dst_h, e_h, emax_h, xl_h, nump_h,
          srcv, dstv, dsc, xls, ev, emv, contrib, shared,
          semA0, semA1, semB0, semB1, semS0, semS1):
        semA = (semA0, semA1)
        semB = (semB0, semB1)
        semS = (semS0, semS1)
        cid = lax.axis_index("c")
        sid = lax.axis_index("s")
        wid = cid * 16 + sid
        base_w = wid * e_per_w
        npt = N_PAD // 16
        lane = _lane()
        zerov = lane.astype(jnp.float32) * 0.0

        def zero_rows(i, _):
            r = i // 9
            col = (i % 9) * 16
            contrib[0, r, pl.ds(col, 16)] = zerov
            return 0
        lax.fori_loop(0, BB * 9, zero_rows, 0)

        for j in range(npt // 40):
            pltpu.sync_copy(
                contrib.at[0, pl.ds(0, 40)],
                shared.at[pl.ds(sid * npt + j * 40, 40)])
        plsc.subcore_barrier()

        def idx_issue(b, g):
            base = base_w + g * BB
            pltpu.async_copy(src_h.at[pl.ds(base, BB)], srcv.at[b], semA[b])
            pltpu.async_copy(dst_h.at[pl.ds(base, BB)], dstv.at[b], semA[b])
            pltpu.async_copy(e_h.at[pl.ds(base, BB)], ev.at[b], semA[b])

        def idx_wait(b, g):
            base = base_w + g * BB
            pltpu.make_async_copy(src_h.at[pl.ds(base, BB)], srcv.at[b],
                                  semA[b]).wait()
            pltpu.make_async_copy(dst_h.at[pl.ds(base, BB)], dstv.at[b],
                                  semA[b]).wait()
            pltpu.make_async_copy(e_h.at[pl.ds(base, BB)], ev.at[b],
                                  semA[b]).wait()

        def gath_issue(b):
            pltpu.async_copy(xl_h.at[srcv.at[b]], xls.at[b], semB[b])
            pltpu.async_copy(emax_h.at[dstv.at[b]], emv.at[b], semB[b])

        def gath_wait(b):
            pltpu.make_async_copy(xl_h.at[srcv.at[b]], xls.at[b],
                                  semB[b]).wait()
            pltpu.make_async_copy(emax_h.at[dstv.at[b]], emv.at[b],
                                  semB[b]).wait()

        def scat_issue(b):
            def cp(i, _):
                dsc[b, pl.ds(i * 16, 16)] = dstv[b, pl.ds(i * 16, 16)]
                return 0
            lax.fori_loop(0, BB // 16, cp, 0)
            pltpu.async_copy(contrib.at[b], shared.at[dsc.at[b]], semS[b],
                             add=True)

        def scat_wait(b):
            pltpu.make_async_copy(contrib.at[b], shared.at[dsc.at[b]],
                                  semS[b]).wait()

        pltpu.sync_copy(src_h.at[pl.ds(base_w, BB)], srcv.at[0])
        pltpu.sync_copy(dst_h.at[pl.ds(base_w, BB)], dstv.at[0])
        pltpu.sync_copy(e_h.at[pl.ds(base_w, BB)], ev.at[0])
        pltpu.sync_copy(src_h.at[pl.ds(base_w + BB, BB)], srcv.at[1])
        pltpu.sync_copy(dst_h.at[pl.ds(base_w + BB, BB)], dstv.at[1])
        pltpu.sync_copy(e_h.at[pl.ds(base_w + BB, BB)], ev.at[1])
        gath_issue(0)
        gath_issue(1)

        def compute(b):
            def edge_body(eb, _):
                ex = jnp.exp(ev[b, eb, pl.ds(0, 16)]
                             - emv[b, eb, pl.ds(0, 16)])
                contrib[b, eb, pl.ds(0, 16)] = ex
                for h in range(H):
                    sp = _take16(ex, (lane & 0) + h)
                    contrib[b, eb, pl.ds(16 + h * C, C)] = (
                        sp * xls[b, eb, pl.ds(h * C, C)])
                return 0
            lax.fori_loop(0, BB, edge_body, 0)

        def outer(gg, _):
            for b in (0, 1):
                g = gg * 2 + b
                gath_wait(b)

                @pl.when(gg >= 1)
                def _():
                    scat_wait(b)

                compute(b)
                scat_issue(b)

                @pl.when(g + 2 < nb)
                def _():
                    idx_issue(b, g + 2)

                bb = 1 - b

                @pl.when(jnp.logical_and(g >= 1, g + 1 < nb))
                def _():
                    idx_wait(bb, g + 1)
                    gath_issue(bb)
            return 0

        lax.fori_loop(0, nb // 2, outer, 0)

        scat_wait(0)
        scat_wait(1)
        plsc.subcore_barrier()

        for j in range(npt // 40):
            row = sid * npt + j * 40
            pltpu.sync_copy(shared.at[pl.ds(row, 40)],
                            contrib.at[0, pl.ds(0, 40)])
            pltpu.sync_copy(contrib.at[0, pl.ds(0, 40)],
                            nump_h.at[cid, pl.ds(row, 40)])

    return k(src, dst, e, emax, xl)


def kernel(x, edges_idx, Wl1, bl1, Wr1, br1, att1, b1,
           Wl2, bl2, Wr2, br2, att2, b2):
    N = x.shape[0]
    E = edges_idx.shape[1]
    e_tot = E + N
    per_round = NW * 384
    e_pad = -(-e_tot // per_round) * per_round
    e_per_w = e_pad // NW

    loop = jnp.arange(N, dtype=edges_idx.dtype)
    ei = jnp.concatenate([edges_idx, jnp.stack([loop, loop])], axis=1)
    pad = jnp.full((2, e_pad - e_tot), N, dtype=edges_idx.dtype)
    ei = jnp.concatenate([ei, pad], axis=1)
    src, dst = ei[0], ei[1]

    xp = jnp.zeros((N_PAD, D), jnp.float32).at[:N].set(x)

    xl1, xr1 = _mm(xp, Wl1, bl1, Wr1, br1)
    e1, maxp1 = _pass_a(src, dst, xl1, xr1, att1, e_per_w)
    emax1 = _maxcomb(maxp1)
    nump1 = _pass_b(src, dst, e1, emax1, xl1, e_per_w)
    xl2, xr2 = _fin_mm(nump1, b1, Wl2, bl2, Wr2, br2)

    e2, maxp2 = _pass_a(src, dst, xl2, xr2, att2, e_per_w)
    emax2 = _maxcomb(maxp2)
    nump2 = _pass_b(src, dst, e2, emax2, xl2, e_per_w)
    out = _fin(nump2, b2)

    return out[:N]
